# Initial kernel scaffold; baseline (speedup 1.0000x reference)
import jax
import jax.numpy as jnp
from jax.experimental import pallas as pl


def kernel(x, edge_index, W_l1, b_l1, W_r1, W_l2, b_l2, W_r2):
    raise NotImplementedError("write your pallas kernel here")


# speedup vs baseline: 7.1819x; 7.1819x over previous
import functools

import jax
import jax.numpy as jnp
from jax import lax
from jax.experimental import pallas as pl
from jax.experimental.pallas import tpu as pltpu
from jax.experimental.pallas import tpu_sc as plsc

N = 10000
E = 320000
D_IN = 128
D_HID = 16
N_CLASSES = 40

NC, NS = 2, 16
TILES = NC * NS
EPT = E // TILES
CHUNK = 80
NCHUNKS = EPT // CHUNK
RPT = 624
RPT_LAST = N - 15 * RPT

_BLK = 2000


def _make_seg_sum(D):
    mesh = plsc.VectorSubcoreMesh(core_axis_name="c", subcore_axis_name="s")

    @functools.partial(
        pl.kernel,
        out_type=jax.ShapeDtypeStruct((NC * N, D), jnp.float32),
        mesh=mesh,
        scratch_types=[
            pltpu.VMEM_SHARED((N, D), jnp.float32),
            pltpu.VMEM((CHUNK,), jnp.int32),
            pltpu.VMEM((CHUNK,), jnp.int32),
            pltpu.VMEM((CHUNK, D), jnp.float32),
            pltpu.SemaphoreType.DMA,
        ],
        compiler_params=pltpu.CompilerParams(use_tc_tiling_on_sc=False),
    )
    def seg_sum(vals_hbm, src_hbm, dst_hbm, zeros_hbm, out_hbm,
                acc, sidx, didx, rows, sem):
        c = lax.axis_index("c")
        s = lax.axis_index("s")
        base = (c * NS + s) * EPT

        @pl.when(s == 0)
        def _zero():
            pltpu.sync_copy(zeros_hbm, acc)

        plsc.subcore_barrier()

        def body(i, carry):
            off = base + i * CHUNK
            pltpu.sync_copy(src_hbm.at[pl.ds(off, CHUNK)], sidx)
            pltpu.sync_copy(dst_hbm.at[pl.ds(off, CHUNK)], didx)
            pltpu.async_copy(vals_hbm.at[sidx], rows, sem).wait()
            pltpu.sync_copy(rows, acc.at[didx], add=True)
            return carry

        lax.fori_loop(0, NCHUNKS, body, 0)

        plsc.subcore_barrier()

        @pl.when(s < NS - 1)
        def _copy_out():
            pltpu.sync_copy(acc.at[pl.ds(s * RPT, RPT)],
                            out_hbm.at[pl.ds(c * N + s * RPT, RPT)])

        @pl.when(s == NS - 1)
        def _copy_out_last():
            pltpu.sync_copy(acc.at[pl.ds(15 * RPT, RPT_LAST)],
                            out_hbm.at[pl.ds(c * N + 15 * RPT, RPT_LAST)])

    return seg_sum


_seg32 = _make_seg_sum(32)
_seg16 = _make_seg_sum(16)


def _mm1_body(x_ref, w1_ref, e1_ref, wr_ref, p1_ref, r1_ref):
    xb = x_ref[...]
    p1_ref[...] = jnp.dot(xb, w1_ref[...],
                          preferred_element_type=jnp.float32) + e1_ref[...]
    r1_ref[...] = jnp.dot(xb, wr_ref[...], preferred_element_type=jnp.float32)


def _mm1(x, w1ext, e16, wr1t):
    return pl.pallas_call(
        _mm1_body,
        grid=(N // _BLK,),
        in_specs=[
            pl.BlockSpec((_BLK, D_IN), lambda i: (i, 0)),
            pl.BlockSpec((D_IN, 32), lambda i: (0, 0)),
            pl.BlockSpec((1, 32), lambda i: (0, 0)),
            pl.BlockSpec((D_IN, D_HID), lambda i: (0, 0)),
        ],
        out_specs=[
            pl.BlockSpec((_BLK, 32), lambda i: (i, 0)),
            pl.BlockSpec((_BLK, D_HID), lambda i: (i, 0)),
        ],
        out_shape=[
            jax.ShapeDtypeStruct((N, 32), jnp.float32),
            jax.ShapeDtypeStruct((N, D_HID), jnp.float32),
        ],
    )(x, w1ext, e16, wr1t)


def _h_body(p_ref, r1_ref, b1_ref, h_ref, dinv_ref):
    pv = p_ref[0] + p_ref[1]
    agg = pv[:, :D_HID]
    deg = pv[:, D_HID:D_HID + 1]
    dinv = 1.0 / jnp.maximum(deg, 1.0)
    h_ref[...] = jnp.maximum(agg * dinv + b1_ref[...] + r1_ref[...], 0.0)
    dinv_ref[...] = dinv


def _h_stage(P, r1, b1):
    return pl.pallas_call(
        _h_body,
        grid=(N // _BLK,),
        in_specs=[
            pl.BlockSpec((2, _BLK, 32), lambda i: (0, i, 0)),
            pl.BlockSpec((_BLK, D_HID), lambda i: (i, 0)),
            pl.BlockSpec((1, D_HID), lambda i: (0, 0)),
        ],
        out_specs=[
            pl.BlockSpec((_BLK, D_HID), lambda i: (i, 0)),
            pl.BlockSpec((_BLK, 1), lambda i: (i, 0)),
        ],
        out_shape=[
            jax.ShapeDtypeStruct((N, D_HID), jnp.float32),
            jax.ShapeDtypeStruct((N, 1), jnp.float32),
        ],
    )(P, r1, b1)


def _out_body(q_ref, dinv_ref, h_ref, wl2_ref, wr2_ref, b2_ref, o_ref):
    aggn = (q_ref[0] + q_ref[1]) * dinv_ref[...]
    o_ref[...] = (jnp.dot(aggn, wl2_ref[...], preferred_element_type=jnp.float32)
                  + jnp.dot(h_ref[...], wr2_ref[...],
                            preferred_element_type=jnp.float32)
                  + b2_ref[...])


def _out_stage(Q, dinv, h, wl2t, wr2t, b2):
    return pl.pallas_call(
        _out_body,
        grid=(N // _BLK,),
        in_specs=[
            pl.BlockSpec((2, _BLK, D_HID), lambda i: (0, i, 0)),
            pl.BlockSpec((_BLK, 1), lambda i: (i, 0)),
            pl.BlockSpec((_BLK, D_HID), lambda i: (i, 0)),
            pl.BlockSpec((D_HID, N_CLASSES), lambda i: (0, 0)),
            pl.BlockSpec((D_HID, N_CLASSES), lambda i: (0, 0)),
            pl.BlockSpec((1, N_CLASSES), lambda i: (0, 0)),
        ],
        out_specs=pl.BlockSpec((_BLK, N_CLASSES), lambda i: (i, 0)),
        out_shape=jax.ShapeDtypeStruct((N, N_CLASSES), jnp.float32),
    )(Q, dinv, h, wl2t, wr2t, b2)


def kernel(x, edge_index, W_l1, b_l1, W_r1, W_l2, b_l2, W_r2):
    src = edge_index[0].astype(jnp.int32)
    dst = edge_index[1].astype(jnp.int32)

    w1ext = jnp.concatenate(
        [W_l1.T, jnp.zeros((D_IN, 32 - D_HID), jnp.float32)], axis=1)
    e16 = jnp.zeros((1, 32), jnp.float32).at[0, D_HID].set(1.0)

    p1ext, r1 = _mm1(x, w1ext, e16, W_r1.T)

    P = _seg32(p1ext, src, dst, jnp.zeros((N, 32), jnp.float32))
    h, dinv = _h_stage(P.reshape(2, N, 32), r1, b_l1.reshape(1, D_HID))

    Q = _seg16(h, src, dst, jnp.zeros((N, D_HID), jnp.float32))
    out = _out_stage(Q.reshape(2, N, D_HID), dinv, h,
                     W_l2.T, W_r2.T, b_l2.reshape(1, N_CLASSES))
    return out
